# baseline (speedup 1.0000x reference)
import functools

import jax
import jax.numpy as jnp
from jax.experimental import pallas as pl
from jax.experimental.pallas import tpu as pltpu

K_TOP = 128


def _enc_kernel(x_ref, w_ref, b_ref, o_ref):
    acc = jnp.dot(x_ref[...], w_ref[...], preferred_element_type=jnp.float32)
    o_ref[...] = jnp.maximum(acc + b_ref[...], 0.0)


def _sel_kernel(pre_ref, z_ref, zb_ref, *, br, hidden):
    m = jnp.max(jax.lax.bitcast_convert_type(pre_ref[...], jnp.int32),
                axis=1, keepdims=True)

    def cond(state):
        b, t, cur = state
        return (b >= 0) & ~jnp.all(cur == K_TOP)

    def bbody(state):
        b, t, cur = state
        cand = t | jnp.left_shift(1, b)

        def scan():
            bits = jax.lax.bitcast_convert_type(pre_ref[...], jnp.int32)
            return jnp.sum((bits >= cand).astype(jnp.int32), axis=1,
                           keepdims=True)

        cnt = jax.lax.cond(jnp.any(cand <= m), scan,
                           lambda: jnp.zeros((br, 1), jnp.int32))
        take = cnt >= K_TOP
        return (b - 1, jnp.where(take, cand, t), jnp.where(take, cnt, cur))

    _, t, _ = jax.lax.while_loop(
        cond, bbody,
        (jnp.int32(30), jnp.zeros((br, 1), jnp.int32),
         jnp.full((br, 1), hidden, jnp.int32)))
    blk = pre_ref[...]
    bits = jax.lax.bitcast_convert_type(blk, jnp.int32)
    zc = jnp.where(bits >= t, blk, 0.0)
    z_ref[...] = zc
    zb_ref[...] = zc.astype(jnp.bfloat16)


def _dec_kernel(zb_ref, w_ref, b_ref, o_ref):
    j = pl.program_id(1)

    @pl.when(j == 0)
    def _init():
        o_ref[...] = jnp.broadcast_to(b_ref[...], o_ref.shape)

    o_ref[...] += jnp.dot(zb_ref[...], w_ref[...],
                          preferred_element_type=jnp.float32)


@jax.jit
def kernel(x, W_enc, b_enc, W_dec, b_dec):
    n, d_in = x.shape
    hidden = W_enc.shape[1]

    br = min(1024, n)
    bn = min(512, hidden)
    pre = pl.pallas_call(
        _enc_kernel,
        grid=(n // br, hidden // bn),
        in_specs=[
            pl.BlockSpec((br, d_in), lambda i, h: (i, 0)),
            pl.BlockSpec((d_in, bn), lambda i, h: (0, h)),
            pl.BlockSpec((1, bn), lambda i, h: (0, h)),
        ],
        out_specs=pl.BlockSpec((br, bn), lambda i, h: (i, h)),
        out_shape=jax.ShapeDtypeStruct((n, hidden), jnp.float32),
        compiler_params=pltpu.CompilerParams(
            dimension_semantics=("parallel", "arbitrary")),
    )(x, W_enc, b_enc.reshape(1, hidden))

    br2 = min(128, n)
    z, zb = pl.pallas_call(
        functools.partial(_sel_kernel, br=br2, hidden=hidden),
        grid=(n // br2,),
        in_specs=[pl.BlockSpec((br2, hidden), lambda i: (i, 0))],
        out_specs=[
            pl.BlockSpec((br2, hidden), lambda i: (i, 0)),
            pl.BlockSpec((br2, hidden), lambda i: (i, 0)),
        ],
        out_shape=[
            jax.ShapeDtypeStruct((n, hidden), jnp.float32),
            jax.ShapeDtypeStruct((n, hidden), jnp.bfloat16),
        ],
        compiler_params=pltpu.CompilerParams(
            dimension_semantics=("parallel",)),
    )(pre)

    br3 = min(1024, n)
    bh3 = min(512, hidden)
    x_recon = pl.pallas_call(
        _dec_kernel,
        grid=(n // br3, hidden // bh3),
        in_specs=[
            pl.BlockSpec((br3, bh3), lambda i, j: (i, j)),
            pl.BlockSpec((bh3, d_in), lambda i, j: (j, 0)),
            pl.BlockSpec((1, d_in), lambda i, j: (0, 0)),
        ],
        out_specs=pl.BlockSpec((br3, d_in), lambda i, j: (i, 0)),
        out_shape=jax.ShapeDtypeStruct((n, d_in), jnp.float32),
        compiler_params=pltpu.CompilerParams(
            dimension_semantics=("parallel", "arbitrary")),
    )(zb, W_dec.astype(jnp.bfloat16), b_dec.reshape(1, d_in))

    return (z, x_recon)


# speedup vs baseline: 1.0357x; 1.0357x over previous
import functools

import jax
import jax.numpy as jnp
from jax.experimental import pallas as pl
from jax.experimental.pallas import tpu as pltpu

K_TOP = 128


def _enc_kernel(x_ref, w_ref, b_ref, o_ref):
    acc = jnp.dot(x_ref[...], w_ref[...], preferred_element_type=jnp.float32)
    o_ref[...] = jnp.maximum(acc + b_ref[...], 0.0)


def _sel_kernel(pre_ref, t_ref, *, br, hidden):
    m = jnp.max(jax.lax.bitcast_convert_type(pre_ref[...], jnp.int32),
                axis=1, keepdims=True)

    def cond(state):
        b, t, cur = state
        return (b >= 0) & ~jnp.all(cur == K_TOP)

    def bbody(state):
        b, t, cur = state
        cand = t | jnp.left_shift(1, b)

        def scan():
            bits = jax.lax.bitcast_convert_type(pre_ref[...], jnp.int32)
            return jnp.sum((bits >= cand).astype(jnp.int32), axis=1,
                           keepdims=True)

        cnt = jax.lax.cond(jnp.any(cand <= m), scan,
                           lambda: jnp.zeros((br, 1), jnp.int32))
        take = cnt >= K_TOP
        return (b - 1, jnp.where(take, cand, t), jnp.where(take, cnt, cur))

    _, t, _ = jax.lax.while_loop(
        cond, bbody,
        (jnp.int32(30), jnp.zeros((br, 1), jnp.int32),
         jnp.full((br, 1), hidden, jnp.int32)))
    t_ref[...] = jnp.broadcast_to(t, t_ref.shape)


def _dec_kernel(pre_ref, t_ref, w_ref, b_ref, z_ref, o_ref):
    j = pl.program_id(1)
    t = t_ref[:, :1]
    blk = pre_ref[...]
    bits = jax.lax.bitcast_convert_type(blk, jnp.int32)
    zc = jnp.where(bits >= t, blk, 0.0)
    z_ref[...] = zc

    @pl.when(j == 0)
    def _init():
        o_ref[...] = jnp.broadcast_to(b_ref[...], o_ref.shape)

    o_ref[...] += jnp.dot(zc.astype(jnp.bfloat16), w_ref[...],
                          preferred_element_type=jnp.float32)


@jax.jit
def kernel(x, W_enc, b_enc, W_dec, b_dec):
    n, d_in = x.shape
    hidden = W_enc.shape[1]

    br = min(1024, n)
    bn = min(512, hidden)
    pre = pl.pallas_call(
        _enc_kernel,
        grid=(n // br, hidden // bn),
        in_specs=[
            pl.BlockSpec((br, d_in), lambda i, h: (i, 0)),
            pl.BlockSpec((d_in, bn), lambda i, h: (0, h)),
            pl.BlockSpec((1, bn), lambda i, h: (0, h)),
        ],
        out_specs=pl.BlockSpec((br, bn), lambda i, h: (i, h)),
        out_shape=jax.ShapeDtypeStruct((n, hidden), jnp.float32),
        compiler_params=pltpu.CompilerParams(
            dimension_semantics=("parallel", "arbitrary")),
    )(x, W_enc, b_enc.reshape(1, hidden))

    br2 = min(256, n)
    thr = pl.pallas_call(
        functools.partial(_sel_kernel, br=br2, hidden=hidden),
        grid=(n // br2,),
        in_specs=[pl.BlockSpec((br2, hidden), lambda i: (i, 0))],
        out_specs=pl.BlockSpec((br2, 128), lambda i: (i, 0)),
        out_shape=jax.ShapeDtypeStruct((n, 128), jnp.int32),
        compiler_params=pltpu.CompilerParams(
            dimension_semantics=("parallel",)),
    )(pre)

    br3 = min(1024, n)
    bh3 = min(512, hidden)
    z, x_recon = pl.pallas_call(
        _dec_kernel,
        grid=(n // br3, hidden // bh3),
        in_specs=[
            pl.BlockSpec((br3, bh3), lambda i, j: (i, j)),
            pl.BlockSpec((br3, 128), lambda i, j: (i, 0)),
            pl.BlockSpec((bh3, d_in), lambda i, j: (j, 0)),
            pl.BlockSpec((1, d_in), lambda i, j: (0, 0)),
        ],
        out_specs=[
            pl.BlockSpec((br3, bh3), lambda i, j: (i, j)),
            pl.BlockSpec((br3, d_in), lambda i, j: (i, 0)),
        ],
        out_shape=[
            jax.ShapeDtypeStruct((n, hidden), jnp.float32),
            jax.ShapeDtypeStruct((n, d_in), jnp.float32),
        ],
        compiler_params=pltpu.CompilerParams(
            dimension_semantics=("parallel", "arbitrary")),
    )(pre, thr, W_dec.astype(jnp.bfloat16), b_dec.reshape(1, d_in))

    return (z, x_recon)
